# Initial kernel scaffold; baseline (speedup 1.0000x reference)
import jax
import jax.numpy as jnp
from jax.experimental import pallas as pl


def kernel(x):
    raise NotImplementedError("write your pallas kernel here")


# speedup vs baseline: 81.2744x; 81.2744x over previous
import functools

import jax
import jax.numpy as jnp
from jax.experimental import pallas as pl

_B, _N = 128, 32768
_K = int(_N * (1.0 - 0.3))
_BLK_R = 8
_MININT = -(2**31)


def _tc_body(x_ref, o_ref):
    xv = x_ref[...]
    bits = jax.lax.bitcast_convert_type(xv, jnp.int32)
    key = bits ^ (jax.lax.shift_right_arithmetic(bits, 31) & jnp.int32(0x7FFFFFFF))

    def step(i, ub):
        b = 31 - i
        trial = ub | (jnp.int32(1) << b)
        thr = trial ^ _MININT
        cnt = jnp.sum((key >= thr).astype(jnp.int32), axis=1, keepdims=True)
        return jnp.where(cnt >= _K, trial, ub)

    ub = jax.lax.fori_loop(0, 32, step, jnp.zeros((_BLK_R, 1), jnp.int32))
    thr = ub ^ _MININT
    o_ref[...] = jnp.where(key >= thr, xv, 0.0)


@jax.jit
def kernel(x):
    return pl.pallas_call(
        _tc_body,
        grid=(_B // _BLK_R,),
        in_specs=[pl.BlockSpec((_BLK_R, _N), lambda i: (i, 0))],
        out_specs=pl.BlockSpec((_BLK_R, _N), lambda i: (i, 0)),
        out_shape=jax.ShapeDtypeStruct((_B, _N), jnp.float32),
    )(x)
